# baseline (speedup 1.0000x reference)
import functools

import jax
import jax.numpy as jnp
from jax import lax
from jax.experimental import pallas as pl
from jax.experimental.pallas import tpu as pltpu
from jax.experimental.pallas import tpu_sc as plsc

N = 10000
D = 128
E = 320000
NC = 2
NS = 16
NW = NC * NS
CHUNK = 128
NCHUNK = E // CHUNK
CPW = 80
PADC = NW * CPW
PAD_E = PADC * CHUNK - E
RT_MAIN = 640
RT_LAST = N - 15 * RT_MAIN
ZROWS = 32
HALF = CPW // 2


def _make_segment_sum():
  mesh = plsc.VectorSubcoreMesh(
      core_axis_name="c", subcore_axis_name="s",
      num_cores=NC, num_subcores=NS)

  @functools.partial(
      pl.kernel,
      out_type=jax.ShapeDtypeStruct((NC, N, D), jnp.float32),
      mesh=mesh,
      scratch_types=[
          pltpu.VMEM((HALF, CHUNK), jnp.int32),
          pltpu.VMEM((HALF, CHUNK), jnp.int32),
          pltpu.VMEM((2, CHUNK, D), jnp.float32),
          pltpu.VMEM((ZROWS, D), jnp.float32),
          pltpu.VMEM_SHARED((N + 8, D), jnp.float32),
          pltpu.SemaphoreType.DMA,
          pltpu.SemaphoreType.DMA,
      ],
  )
  def segsum(h_hbm, src_hbm, dst_hbm, out_hbm,
             sidx, didx, rows, zbuf, acc, gsem0, gsem1):
    c = lax.axis_index("c")
    s = lax.axis_index("s")
    wid = s * NC + c
    gsem = (gsem0, gsem1)

    zv = jnp.zeros((16,), jnp.float32)

    @pl.loop(0, ZROWS)
    def _(r):
      @pl.loop(0, D // 16)
      def _(j):
        zbuf[r, pl.ds(j * 16, 16)] = zv

    base = s * RT_MAIN

    @pl.when(s < NS - 1)
    def _():
      for j in range(RT_MAIN // ZROWS):
        pltpu.sync_copy(zbuf, acc.at[pl.ds(base + j * ZROWS, ZROWS)])

    @pl.when(s == NS - 1)
    def _():
      zlast = N + 8 - 15 * RT_MAIN
      for j in range(zlast // ZROWS):
        pltpu.sync_copy(zbuf, acc.at[pl.ds(base + j * ZROWS, ZROWS)])
      rem = zlast % ZROWS
      if rem:
        pltpu.sync_copy(zbuf.at[pl.ds(0, rem)],
                        acc.at[pl.ds(base + zlast - rem, rem)])

    plsc.subcore_barrier()

    c0 = wid * CPW

    def start(t, b):
      pltpu.async_copy(h_hbm.at[sidx.at[t]], rows.at[b], gsem[b])

    def consume(t, b):
      pltpu.make_async_copy(h_hbm.at[sidx.at[t]], rows.at[b], gsem[b]).wait()
      pltpu.sync_copy(rows.at[b], acc.at[didx.at[t]], add=True)

    for h0 in (0, HALF):
      pltpu.sync_copy(src_hbm.at[pl.ds(c0 + h0, HALF)], sidx)
      pltpu.sync_copy(dst_hbm.at[pl.ds(c0 + h0, HALF)], didx)
      start(0, 0)

      @pl.loop(0, HALF, step=2)
      def _(t):
        start(t + 1, 1)
        consume(t, 0)

        @pl.when(t + 2 < HALF)
        def _():
          start(t + 2, 0)

        consume(t + 1, 1)

    plsc.subcore_barrier()

    @pl.when(s < NS - 1)
    def _():
      sl = pl.ds(base, RT_MAIN)
      pltpu.sync_copy(acc.at[sl], out_hbm.at[c, sl])

    @pl.when(s == NS - 1)
    def _():
      sl = pl.ds(base, RT_LAST)
      pltpu.sync_copy(acc.at[sl], out_hbm.at[c, sl])

  return segsum


@functools.lru_cache(maxsize=1)
def _segment_sum_fn():
  return _make_segment_sum()


def _pad_edges(src, dst):
  zpad = jnp.zeros((PAD_E,), jnp.int32)
  tpad = N + (jnp.arange(PAD_E, dtype=jnp.int32) % 8)
  srcp = jnp.concatenate([src, zpad]).reshape(PADC, CHUNK)
  dstp = jnp.concatenate([dst, tpad]).reshape(PADC, CHUNK)
  return srcp, dstp


def _segment_sum(h, src, dst):
  return _segment_sum_fn()(h, src, dst)


def _ln(t, g, b):
  m = jnp.mean(t, axis=-1, keepdims=True)
  v = jnp.mean((t - m) ** 2, axis=-1, keepdims=True)
  return (t - m) * lax.rsqrt(v + 1e-5) * g + b


def _dense_body(parts_ref, x_ref, norm_ref, w1_ref, b1_ref, g1_ref, be1_ref,
                w2_ref, b2_ref, g2_ref, be2_ref, out_ref, *, final):
  x = x_ref[...]
  agg = parts_ref[0] + parts_ref[1]
  hm = (agg - x) * norm_ref[...]
  t = (jnp.dot(hm, w1_ref[0:D, :], preferred_element_type=jnp.float32)
       + jnp.dot(x, w1_ref[D:2 * D, :], preferred_element_type=jnp.float32)
       + b1_ref[...])
  t = jnp.maximum(_ln(t, g1_ref[...], be1_ref[...]), 0.0)
  t = jnp.dot(t, w2_ref[...], preferred_element_type=jnp.float32) + b2_ref[...]
  if not final:
    t = jnp.maximum(_ln(t, g2_ref[...], be2_ref[...]), 0.0)
  out_ref[...] = t


def _dense(parts, x, norm, w1, b1, g1, be1, w2, b2, g2, be2, *, final):
  R = 1000
  grid = (N // R,)
  row = lambda i: (i, 0)
  full = lambda i: (0, 0)
  return pl.pallas_call(
      functools.partial(_dense_body, final=final),
      grid=grid,
      in_specs=[
          pl.BlockSpec((NC, R, D), lambda i: (0, i, 0)),
          pl.BlockSpec((R, D), row),
          pl.BlockSpec((R, 1), row),
          pl.BlockSpec((2 * D, D), full),
          pl.BlockSpec((1, D), full),
          pl.BlockSpec((1, D), full),
          pl.BlockSpec((1, D), full),
          pl.BlockSpec((D, D), full),
          pl.BlockSpec((1, D), full),
          pl.BlockSpec((1, D), full),
          pl.BlockSpec((1, D), full),
      ],
      out_specs=pl.BlockSpec((R, D), row),
      out_shape=jax.ShapeDtypeStruct((N, D), jnp.float32),
  )(parts, x, norm, w1, b1, g1, be1, w2, b2, g2, be2)


def kernel(x, edge_index, norm,
           W1_0, b1_0, g1_0, be1_0, W2_0, b2_0, g2_0, be2_0,
           W1_1, b1_1, g1_1, be1_1, W2_1, b2_1):
  src = edge_index[0].astype(jnp.int32)
  dst = edge_index[1].astype(jnp.int32)
  src, dst = _pad_edges(src, dst)
  r2 = lambda v: v.reshape(1, D)

  parts = _segment_sum(x, src, dst)
  h = _dense(parts, x, norm, W1_0, r2(b1_0), r2(g1_0), r2(be1_0),
             W2_0, r2(b2_0), r2(g2_0), r2(be2_0), final=False)
  parts = _segment_sum(h, src, dst)
  out = _dense(parts, h, norm, W1_1, r2(b1_1), r2(g1_1), r2(be1_1),
               W2_1, r2(b2_1), r2(g1_1), r2(be1_1), final=True)
  return out


# speedup vs baseline: 3.2568x; 3.2568x over previous
import functools

import jax
import jax.numpy as jnp
from jax import lax
from jax.experimental import pallas as pl
from jax.experimental.pallas import tpu as pltpu
from jax.experimental.pallas import tpu_sc as plsc

N = 10000
D = 128
E = 320000
NC = 2
NS = 16
NW = NC * NS
CHUNK = 128
NCHUNK = E // CHUNK
CPW = 80
PADC = NW * CPW
PAD_E = PADC * CHUNK - E
RT_MAIN = 640
RT_LAST = N - 15 * RT_MAIN
ZROWS = 32
HALF = CPW // 2
TRASH = 256


def _make_segment_sum():
  mesh = plsc.VectorSubcoreMesh(
      core_axis_name="c", subcore_axis_name="s",
      num_cores=NC, num_subcores=NS)

  @functools.partial(
      pl.kernel,
      out_type=jax.ShapeDtypeStruct((NC, N, D), jnp.float32),
      mesh=mesh,
      scratch_types=[
          pltpu.VMEM((HALF, CHUNK), jnp.int32),
          pltpu.VMEM((HALF, CHUNK), jnp.int32),
          pltpu.VMEM((2, CHUNK, D), jnp.float32),
          pltpu.VMEM((ZROWS, D), jnp.float32),
          pltpu.VMEM_SHARED((N + TRASH, D), jnp.float32),
          pltpu.SemaphoreType.DMA,
          pltpu.SemaphoreType.DMA,
      ],
  )
  def segsum(h_hbm, src_hbm, dst_hbm, out_hbm,
             sidx, didx, rows, zbuf, acc, gsem0, gsem1):
    c = lax.axis_index("c")
    s = lax.axis_index("s")
    wid = s * NC + c
    gsem = (gsem0, gsem1)

    zv = jnp.zeros((16,), jnp.float32)

    @pl.loop(0, ZROWS)
    def _(r):
      @pl.loop(0, D // 16)
      def _(j):
        zbuf[r, pl.ds(j * 16, 16)] = zv

    base = s * RT_MAIN

    @pl.when(s < NS - 1)
    def _():
      for j in range(RT_MAIN // ZROWS):
        pltpu.sync_copy(zbuf, acc.at[pl.ds(base + j * ZROWS, ZROWS)])

    @pl.when(s == NS - 1)
    def _():
      zlast = N + TRASH - 15 * RT_MAIN
      for j in range(zlast // ZROWS):
        pltpu.sync_copy(zbuf, acc.at[pl.ds(base + j * ZROWS, ZROWS)])
      rem = zlast % ZROWS
      if rem:
        pltpu.sync_copy(zbuf.at[pl.ds(0, rem)],
                        acc.at[pl.ds(base + zlast - rem, rem)])

    plsc.subcore_barrier()

    c0 = wid * CPW

    def start(t, b):
      pltpu.async_copy(h_hbm.at[sidx.at[t]], rows.at[b], gsem[b])

    def consume(t, b):
      pltpu.make_async_copy(h_hbm.at[sidx.at[t]], rows.at[b], gsem[b]).wait()
      pltpu.sync_copy(rows.at[b], acc.at[didx.at[t]], add=True)

    for h0 in (0, HALF):
      pltpu.sync_copy(src_hbm.at[pl.ds(c0 + h0, HALF)], sidx)
      pltpu.sync_copy(dst_hbm.at[pl.ds(c0 + h0, HALF)], didx)
      start(0, 0)

      @pl.loop(0, HALF, step=2)
      def _(t):
        start(t + 1, 1)
        consume(t, 0)

        @pl.when(t + 2 < HALF)
        def _():
          start(t + 2, 0)

        consume(t + 1, 1)

    plsc.subcore_barrier()

    @pl.when(s < NS - 1)
    def _():
      sl = pl.ds(base, RT_MAIN)
      pltpu.sync_copy(acc.at[sl], out_hbm.at[c, sl])

    @pl.when(s == NS - 1)
    def _():
      sl = pl.ds(base, RT_LAST)
      pltpu.sync_copy(acc.at[sl], out_hbm.at[c, sl])

  return segsum


@functools.lru_cache(maxsize=1)
def _segment_sum_fn():
  return _make_segment_sum()


def _pad_edges(src, dst):
  ar = jnp.arange(PAD_E, dtype=jnp.int32)
  zpad = ar % N
  tpad = N + (ar % TRASH)
  srcp = jnp.concatenate([src, zpad]).reshape(PADC, CHUNK)
  dstp = jnp.concatenate([dst, tpad]).reshape(PADC, CHUNK)
  return srcp, dstp


def _segment_sum(h, src, dst):
  return _segment_sum_fn()(h, src, dst)


def _ln(t, g, b):
  m = jnp.mean(t, axis=-1, keepdims=True)
  v = jnp.mean((t - m) ** 2, axis=-1, keepdims=True)
  return (t - m) * lax.rsqrt(v + 1e-5) * g + b


def _dense_body(parts_ref, x_ref, norm_ref, w1_ref, b1_ref, g1_ref, be1_ref,
                w2_ref, b2_ref, g2_ref, be2_ref, out_ref, *, final):
  x = x_ref[...]
  agg = parts_ref[0] + parts_ref[1]
  hm = (agg - x) * norm_ref[...]
  t = (jnp.dot(hm, w1_ref[0:D, :], preferred_element_type=jnp.float32)
       + jnp.dot(x, w1_ref[D:2 * D, :], preferred_element_type=jnp.float32)
       + b1_ref[...])
  t = jnp.maximum(_ln(t, g1_ref[...], be1_ref[...]), 0.0)
  t = jnp.dot(t, w2_ref[...], preferred_element_type=jnp.float32) + b2_ref[...]
  if not final:
    t = jnp.maximum(_ln(t, g2_ref[...], be2_ref[...]), 0.0)
  out_ref[...] = t


def _dense(parts, x, norm, w1, b1, g1, be1, w2, b2, g2, be2, *, final):
  R = 1000
  grid = (N // R,)
  row = lambda i: (i, 0)
  full = lambda i: (0, 0)
  return pl.pallas_call(
      functools.partial(_dense_body, final=final),
      grid=grid,
      in_specs=[
          pl.BlockSpec((NC, R, D), lambda i: (0, i, 0)),
          pl.BlockSpec((R, D), row),
          pl.BlockSpec((R, 1), row),
          pl.BlockSpec((2 * D, D), full),
          pl.BlockSpec((1, D), full),
          pl.BlockSpec((1, D), full),
          pl.BlockSpec((1, D), full),
          pl.BlockSpec((D, D), full),
          pl.BlockSpec((1, D), full),
          pl.BlockSpec((1, D), full),
          pl.BlockSpec((1, D), full),
      ],
      out_specs=pl.BlockSpec((R, D), row),
      out_shape=jax.ShapeDtypeStruct((N, D), jnp.float32),
  )(parts, x, norm, w1, b1, g1, be1, w2, b2, g2, be2)


def kernel(x, edge_index, norm,
           W1_0, b1_0, g1_0, be1_0, W2_0, b2_0, g2_0, be2_0,
           W1_1, b1_1, g1_1, be1_1, W2_1, b2_1):
  src = edge_index[0].astype(jnp.int32)
  dst = edge_index[1].astype(jnp.int32)
  src, dst = _pad_edges(src, dst)
  r2 = lambda v: v.reshape(1, D)

  parts = _segment_sum(x, src, dst)
  h = _dense(parts, x, norm, W1_0, r2(b1_0), r2(g1_0), r2(be1_0),
             W2_0, r2(b2_0), r2(g2_0), r2(be2_0), final=False)
  parts = _segment_sum(h, src, dst)
  out = _dense(parts, h, norm, W1_1, r2(b1_1), r2(g1_1), r2(be1_1),
               W2_1, r2(b2_1), r2(g1_1), r2(be1_1), final=True)
  return out
